# Initial kernel scaffold; baseline (speedup 1.0000x reference)
import jax
import jax.numpy as jnp
from jax.experimental import pallas as pl


def kernel(x, tables):
    raise NotImplementedError("write your pallas kernel here")


# speedup vs baseline: 1.0693x; 1.0693x over previous
import functools

import jax
import jax.numpy as jnp
from jax import lax
from jax.experimental import pallas as pl
from jax.experimental.pallas import tpu as pltpu
from jax.experimental.pallas import tpu_sc as plsc

F = 26
V = 100000
D = 32
B = 16384
L = 16
NC, NS = 2, 16
NW = NC * NS
BPW = B // NW
CHUNK = 128
NCHUNK = (BPW * F) // CHUNK


def _body(x_hbm, tbl_hbm, out_hbm, xv, idxq, buf, acc, sem):
    wid = lax.axis_index("s") * NC + lax.axis_index("c")
    base = wid * BPW

    pltpu.sync_copy(x_hbm.at[pl.ds(base * F, BPW * F)], xv)

    rows_per_chunk = CHUNK // L
    for f in range(F):
        def bidx(j, _, f=f):
            vals = xv[pl.ds(f * BPW + j * L, L)] + (f * V)
            row = f * (BPW // CHUNK) + j // rows_per_chunk
            col = (j % rows_per_chunk) * L
            idxq[row, pl.ds(col, L)] = vals
            return 0
        lax.fori_loop(0, BPW // L, bidx, 0)

    zv = jnp.zeros((L,), jnp.float32)
    def zb(r, _):
        acc[r, pl.ds(0, L)] = zv
        acc[r, pl.ds(L, L)] = zv
        return 0
    lax.fori_loop(0, BPW, zb, 0)

    def cb(c, _):
        pltpu.async_copy(tbl_hbm.at[idxq.at[c]], buf, sem).wait()
        r0 = (c % (BPW // CHUNK)) * CHUNK
        def rb(r8, _):
            for rr in range(8):
                r = r8 * 8 + rr
                plsc.addupdate(acc.at[r0 + r, pl.ds(0, L)], buf[r, pl.ds(0, L)])
                plsc.addupdate(acc.at[r0 + r, pl.ds(L, L)], buf[r, pl.ds(L, L)])
            return 0
        lax.fori_loop(0, CHUNK // 8, rb, 0)
        return 0
    lax.fori_loop(0, NCHUNK, cb, 0)

    pltpu.sync_copy(acc, out_hbm.at[pl.ds(base, BPW)])


@jax.jit
def _run(x_flat, tbl_flat):
    mesh = plsc.VectorSubcoreMesh(core_axis_name="c", subcore_axis_name="s")
    return pl.kernel(
        _body,
        out_type=jax.ShapeDtypeStruct((B, D), jnp.float32),
        mesh=mesh,
        scratch_types=[
            pltpu.VMEM((BPW * F,), jnp.int32),
            pltpu.VMEM((NCHUNK, CHUNK), jnp.int32),
            pltpu.VMEM((CHUNK, D), jnp.float32),
            pltpu.VMEM((BPW, D), jnp.float32),
            pltpu.SemaphoreType.DMA,
        ],
        compiler_params=pltpu.CompilerParams(use_tc_tiling_on_sc=False),
    )(x_flat, tbl_flat)


def kernel(x, tables):
    xt = x.reshape(NW, BPW, F).transpose(0, 2, 1).reshape(B * F)
    return _run(xt, tables.reshape(F * V, D))


# speedup vs baseline: 1.0713x; 1.0019x over previous
import jax
import jax.numpy as jnp
from jax import lax
from jax.experimental import pallas as pl
from jax.experimental.pallas import tpu as pltpu
from jax.experimental.pallas import tpu_sc as plsc

F = 26
V = 100000
D = 32
B = 16384
L = 16
NC, NS = 2, 16
NW = NC * NS
BPW = B // NW
CHUNK = 128
CPF = BPW // CHUNK
NCHUNK = CPF * F


def _body(xt_hbm, tbl_hbm, out_hbm, idxq, buf, acc, sem):
    wid = lax.axis_index("s") * NC + lax.axis_index("c")

    pltpu.sync_copy(xt_hbm.at[pl.ds(wid * NCHUNK, NCHUNK)], idxq)

    zv = jnp.zeros((L,), jnp.float32)
    def zb(r, _):
        acc[r, pl.ds(0, L)] = zv
        acc[r, pl.ds(L, L)] = zv
        return 0
    lax.fori_loop(0, BPW, zb, 0)

    for f in range(F):
        def cb(k, _, f=f):
            pltpu.async_copy(tbl_hbm.at[f].at[idxq.at[f * CPF + k]], buf, sem).wait()
            r0 = k * CHUNK
            def rb(r8, _):
                for rr in range(8):
                    r = r8 * 8 + rr
                    plsc.addupdate(acc.at[r0 + r, pl.ds(0, L)], buf[r, pl.ds(0, L)])
                    plsc.addupdate(acc.at[r0 + r, pl.ds(L, L)], buf[r, pl.ds(L, L)])
                return 0
            lax.fori_loop(0, CHUNK // 8, rb, 0)
            return 0
        lax.fori_loop(0, CPF, cb, 0)

    pltpu.sync_copy(acc, out_hbm.at[pl.ds(wid * BPW, BPW)])


@jax.jit
def _run(xt, tables):
    mesh = plsc.VectorSubcoreMesh(core_axis_name="c", subcore_axis_name="s")
    return pl.kernel(
        _body,
        out_type=jax.ShapeDtypeStruct((B, D), jnp.float32),
        mesh=mesh,
        scratch_types=[
            pltpu.VMEM((NCHUNK, CHUNK), jnp.int32),
            pltpu.VMEM((CHUNK, D), jnp.float32),
            pltpu.VMEM((BPW, D), jnp.float32),
            pltpu.SemaphoreType.DMA,
        ],
        compiler_params=pltpu.CompilerParams(use_tc_tiling_on_sc=False),
    )(xt, tables)


def kernel(x, tables):
    xt = x.reshape(NW, BPW, F).transpose(0, 2, 1).reshape(-1, CHUNK)
    return _run(xt, tables)


# speedup vs baseline: 1.1375x; 1.0618x over previous
import jax
import jax.numpy as jnp
from jax import lax
from jax.experimental import pallas as pl
from jax.experimental.pallas import tpu as pltpu
from jax.experimental.pallas import tpu_sc as plsc

F = 26
V = 100000
D = 32
B = 16384
L = 16
NC, NS = 2, 16
NW = NC * NS
BPW = B // NW
CHUNK = 128
CPF = BPW // CHUNK
NCHUNK = CPF * F


def _body(xt_hbm, tbl_hbm, out_hbm, idxq, bufa, bufb, acc, sem):
    wid = lax.axis_index("s") * NC + lax.axis_index("c")

    pltpu.sync_copy(xt_hbm.at[pl.ds(wid * NCHUNK, NCHUNK)], idxq)

    zv = jnp.zeros((L,), jnp.float32)
    def zb(r, _):
        acc[r, pl.ds(0, L)] = zv
        acc[r, pl.ds(L, L)] = zv
        return 0
    lax.fori_loop(0, BPW, zb, 0)

    def fire(c, dst):
        f = c // CPF
        pltpu.async_copy(tbl_hbm.at[f].at[idxq.at[c]], dst, sem)

    def drain(c, src):
        f = c // CPF
        pltpu.make_async_copy(tbl_hbm.at[f].at[idxq.at[c]], src, sem).wait()

    def accum(c, src):
        r0 = (c % CPF) * CHUNK
        def rb(r8, _):
            for rr in range(8):
                r = r8 * 8 + rr
                plsc.addupdate(acc.at[r0 + r, pl.ds(0, L)], src[r, pl.ds(0, L)])
                plsc.addupdate(acc.at[r0 + r, pl.ds(L, L)], src[r, pl.ds(L, L)])
            return 0
        lax.fori_loop(0, CHUNK // 8, rb, 0)

    fire(0, bufa)
    def pb(p, _):
        e = 2 * p
        fire(e + 1, bufb)
        drain(e, bufa)
        accum(e, bufa)
        @pl.when(p < NCHUNK // 2 - 1)
        def _():
            fire(e + 2, bufa)
        drain(e + 1, bufb)
        accum(e + 1, bufb)
        return 0
    lax.fori_loop(0, NCHUNK // 2, pb, 0)

    pltpu.sync_copy(acc, out_hbm.at[pl.ds(wid * BPW, BPW)])


@jax.jit
def _run(xt, tables):
    mesh = plsc.VectorSubcoreMesh(core_axis_name="c", subcore_axis_name="s")
    return pl.kernel(
        _body,
        out_type=jax.ShapeDtypeStruct((B, D), jnp.float32),
        mesh=mesh,
        scratch_types=[
            pltpu.VMEM((NCHUNK, CHUNK), jnp.int32),
            pltpu.VMEM((CHUNK, D), jnp.float32),
            pltpu.VMEM((CHUNK, D), jnp.float32),
            pltpu.VMEM((BPW, D), jnp.float32),
            pltpu.SemaphoreType.DMA,
        ],
        compiler_params=pltpu.CompilerParams(use_tc_tiling_on_sc=False),
    )(xt, tables)


def kernel(x, tables):
    xt = x.reshape(NW, BPW, F).transpose(0, 2, 1).reshape(-1, CHUNK)
    return _run(xt, tables)
